# Initial kernel scaffold; baseline (speedup 1.0000x reference)
import jax
import jax.numpy as jnp
from jax.experimental import pallas as pl


def kernel(x, Wg, We):
    raise NotImplementedError("write your pallas kernel here")


# speedup vs baseline: 1.7882x; 1.7882x over previous
import functools

import jax
import jax.numpy as jnp
from jax import lax
from jax.experimental import pallas as pl
from jax.experimental.pallas import tpu as pltpu
from jax.experimental.pallas import tpu_sc as plsc

E = 8
H = 1024
I = 2048
N = 2048
T = 256
G = N // T + E
PADDED = G * T

NW = 32
TPW = N // NW
CH = 32


def _route_kernel(x_ref, wg_ref, pos_ref, te_ref, valid_ref):
    x = x_ref[...]
    wg = wg_ref[...]
    scores = lax.dot_general(wg, x, (((1,), (1,)), ((), ())),
                             preferred_element_type=jnp.float32)
    e_iota = lax.broadcasted_iota(jnp.int32, (E, N), 0)
    mx = jnp.max(scores, axis=0, keepdims=True)
    idx = jnp.min(jnp.where(scores == mx, e_iota, E), axis=0, keepdims=True)
    onehot = (e_iota == idx).astype(jnp.float32)

    r = lax.broadcasted_iota(jnp.int32, (N, N), 0)
    c = lax.broadcasted_iota(jnp.int32, (N, N), 1)
    tri = (r <= c).astype(jnp.float32)
    csum = lax.dot_general(onehot, tri, (((1,), (0,)), ((), ())),
                           preferred_element_type=jnp.float32)

    counts = csum[:, N - 1:N]
    counts_i = counts.astype(jnp.int32)
    pc_i = ((counts_i + (T - 1)) // T) * T
    pc = pc_i.astype(jnp.float32)

    er = lax.broadcasted_iota(jnp.int32, (E, E), 0)
    ec = lax.broadcasted_iota(jnp.int32, (E, E), 1)
    lo = (ec < er).astype(jnp.float32)
    pc_wide = jnp.broadcast_to(pc, (E, 128))
    poff_wide = lax.dot_general(lo, pc_wide, (((1,), (0,)), ((), ())),
                                preferred_element_type=jnp.float32)
    poff = poff_wide[:, 0:1]

    pos = jnp.sum(onehot * (poff + csum - 1.0), axis=0, keepdims=True)
    pos_ref[...] = pos.astype(jnp.int32)

    gt = (lax.broadcasted_iota(jnp.int32, (1, G), 1) * T).astype(jnp.float32)
    eg = lax.broadcasted_iota(jnp.int32, (E, G), 0)
    covered = jnp.logical_and(poff <= gt, eg >= 1)
    te_raw = jnp.sum(covered.astype(jnp.int32), axis=0, keepdims=True)
    nz = counts > 0.0
    e_id = lax.broadcasted_iota(jnp.int32, (E, 1), 0)
    e_last = jnp.max(jnp.where(nz, e_id, 0), axis=0, keepdims=True)
    te_ref[...] = jnp.minimum(te_raw, e_last)
    total = poff[E - 1:E, 0:1] + pc[E - 1:E, 0:1]
    valid_ref[...] = (gt < total).astype(jnp.int32)


def _mm_kernel(te_ref, valid_ref, xs_ref, we_ref, ys_ref):
    g = pl.program_id(0)

    @pl.when(valid_ref[g] == 1)
    def _():
        ys_ref[...] = lax.dot_general(
            xs_ref[...], we_ref[0],
            (((1,), (1,)), ((), ())),
            preferred_element_type=jnp.float32)


@functools.cache
def _sc_kernels():
    mesh = plsc.VectorSubcoreMesh(core_axis_name="c", subcore_axis_name="s")

    @functools.partial(
        pl.kernel,
        mesh=mesh,
        out_type=jax.ShapeDtypeStruct((PADDED, H), jnp.float32),
        scratch_types=[
            pltpu.VMEM((TPW,), jnp.int32),
            pltpu.VMEM((TPW, H), jnp.float32),
            pltpu.SemaphoreType.DMA,
        ],
    )
    def _dispatch(x_hbm, pos_hbm, xs_hbm, idx_v, rows_v, sem):
        wid = lax.axis_index("s") * 2 + lax.axis_index("c")
        base = wid * TPW
        pltpu.sync_copy(pos_hbm.at[pl.ds(base, TPW)], idx_v)
        pltpu.sync_copy(x_hbm.at[pl.ds(base, TPW)], rows_v)
        pltpu.async_copy(rows_v, xs_hbm.at[idx_v], sem).wait()

    @functools.partial(
        pl.kernel,
        mesh=mesh,
        out_type=jax.ShapeDtypeStruct((N, I), jnp.float32),
        scratch_types=[
            pltpu.VMEM((CH,), jnp.int32),
            pltpu.VMEM((CH, I), jnp.float32),
            pltpu.SemaphoreType.DMA,
        ],
    )
    def _combine(ys_hbm, pos_hbm, out_hbm, idx_v, rows_v, sem):
        wid = lax.axis_index("s") * 2 + lax.axis_index("c")
        base = wid * TPW
        for chunk in range(TPW // CH):
            off = base + chunk * CH
            pltpu.sync_copy(pos_hbm.at[pl.ds(off, CH)], idx_v)
            pltpu.async_copy(ys_hbm.at[idx_v], rows_v, sem).wait()
            pltpu.sync_copy(rows_v, out_hbm.at[pl.ds(off, CH)])

    return _dispatch, _combine


def kernel(x, Wg, We):
    pos2d, te2d, valid2d = pl.pallas_call(
        _route_kernel,
        out_shape=[
            jax.ShapeDtypeStruct((1, N), jnp.int32),
            jax.ShapeDtypeStruct((1, G), jnp.int32),
            jax.ShapeDtypeStruct((1, G), jnp.int32),
        ],
    )(x, Wg)
    pos = pos2d.reshape(N)
    te = te2d.reshape(G)
    valid = valid2d.reshape(G)

    dispatch, combine = _sc_kernels()
    xs = dispatch(x, pos)

    grid_spec = pltpu.PrefetchScalarGridSpec(
        num_scalar_prefetch=2,
        grid=(G,),
        in_specs=[
            pl.BlockSpec((T, H), lambda g, te, valid: (g, 0)),
            pl.BlockSpec((1, I, H), lambda g, te, valid: (te[g], 0, 0)),
        ],
        out_specs=pl.BlockSpec((T, I), lambda g, te, valid: (g, 0)),
    )
    ys = pl.pallas_call(
        _mm_kernel,
        grid_spec=grid_spec,
        out_shape=jax.ShapeDtypeStruct((PADDED, I), jnp.float32),
    )(te, valid, xs, We)

    return combine(ys, pos)


# speedup vs baseline: 1.8589x; 1.0395x over previous
import functools

import jax
import jax.numpy as jnp
from jax import lax
from jax.experimental import pallas as pl
from jax.experimental.pallas import tpu as pltpu
from jax.experimental.pallas import tpu_sc as plsc

E = 8
H = 1024
I = 2048
N = 2048
T = 256
G = N // T + E
PADDED = G * T

NW = 32
TPW = N // NW
CH = 16


def _route_kernel(x_ref, wg_ref, pos_ref, te_ref, valid_ref, row_ref):
    x = x_ref[...]
    wg = wg_ref[...]
    scores = lax.dot_general(wg, x, (((1,), (1,)), ((), ())),
                             preferred_element_type=jnp.float32)
    e_iota = lax.broadcasted_iota(jnp.int32, (E, N), 0)
    mx = jnp.max(scores, axis=0, keepdims=True)
    idx = jnp.min(jnp.where(scores == mx, e_iota, E), axis=0, keepdims=True)
    onehot = (e_iota == idx).astype(jnp.float32)

    r = lax.broadcasted_iota(jnp.int32, (N, N), 0)
    c = lax.broadcasted_iota(jnp.int32, (N, N), 1)
    tri = (r <= c).astype(jnp.float32)
    csum = lax.dot_general(onehot, tri, (((1,), (0,)), ((), ())),
                           preferred_element_type=jnp.float32)

    counts = csum[:, N - 1:N]
    counts_i = counts.astype(jnp.int32)
    pc_i = ((counts_i + (T - 1)) // T) * T
    pc = pc_i.astype(jnp.float32)

    er = lax.broadcasted_iota(jnp.int32, (E, E), 0)
    ec = lax.broadcasted_iota(jnp.int32, (E, E), 1)
    lo = (ec < er).astype(jnp.float32)
    pc_wide = jnp.broadcast_to(pc, (E, 128))
    poff_wide = lax.dot_general(lo, pc_wide, (((1,), (0,)), ((), ())),
                                preferred_element_type=jnp.float32)
    poff = poff_wide[:, 0:1]

    pos = jnp.sum(onehot * (poff + csum - 1.0), axis=0, keepdims=True)
    pos_ref[...] = pos.astype(jnp.int32)

    gt = (lax.broadcasted_iota(jnp.int32, (1, G), 1) * T).astype(jnp.float32)
    eg = lax.broadcasted_iota(jnp.int32, (E, G), 0)
    covered = jnp.logical_and(poff <= gt, eg >= 1)
    te_raw = jnp.sum(covered.astype(jnp.int32), axis=0, keepdims=True)
    nz = counts > 0.0
    e_id = lax.broadcasted_iota(jnp.int32, (E, 1), 0)
    e_last = jnp.max(jnp.where(nz, e_id, 0), axis=0, keepdims=True)
    te_ref[...] = jnp.minimum(te_raw, e_last)
    total = poff[E - 1:E, 0:1] + pc[E - 1:E, 0:1]
    valid_ref[...] = (gt < total).astype(jnp.int32)
    g_iota = lax.broadcasted_iota(jnp.int32, (1, G), 1)
    last_valid = (total.astype(jnp.int32) // T) - 1
    row_ref[...] = jnp.minimum(g_iota, last_valid)


def _mm_kernel(te_ref, valid_ref, row_ref, xs_ref, we_ref, ys_ref):
    g = pl.program_id(0)

    @pl.when(valid_ref[g] == 1)
    def _():
        ys_ref[...] = lax.dot_general(
            xs_ref[...], we_ref[0],
            (((1,), (1,)), ((), ())),
            preferred_element_type=jnp.float32)


@functools.cache
def _sc_kernels():
    mesh = plsc.VectorSubcoreMesh(core_axis_name="c", subcore_axis_name="s")

    DC = TPW // 2

    @functools.partial(
        pl.kernel,
        mesh=mesh,
        out_type=jax.ShapeDtypeStruct((PADDED, H), jnp.float32),
        scratch_types=[
            pltpu.VMEM((2, DC), jnp.int32),
            pltpu.VMEM((DC, H), jnp.float32),
            pltpu.VMEM((DC, H), jnp.float32),
            pltpu.SemaphoreType.DMA,
            pltpu.SemaphoreType.DMA,
            pltpu.SemaphoreType.DMA,
            pltpu.SemaphoreType.DMA,
        ],
    )
    def _dispatch(x_hbm, pos_hbm, xs_hbm, idx_v, rows0, rows1, l0, l1, s0, s1):
        wid = lax.axis_index("s") * 2 + lax.axis_index("c")
        base = wid * TPW
        pltpu.sync_copy(pos_hbm.at[pl.ds(base, DC)], idx_v.at[0])
        pltpu.sync_copy(pos_hbm.at[pl.ds(base + DC, DC)], idx_v.at[1])
        ld0 = pltpu.async_copy(x_hbm.at[pl.ds(base, DC)], rows0, l0)
        ld1 = pltpu.async_copy(x_hbm.at[pl.ds(base + DC, DC)], rows1, l1)
        ld0.wait()
        st0 = pltpu.async_copy(rows0, xs_hbm.at[idx_v.at[0]], s0)
        ld1.wait()
        st1 = pltpu.async_copy(rows1, xs_hbm.at[idx_v.at[1]], s1)
        st0.wait()
        st1.wait()

    NCH = TPW // CH

    @functools.partial(
        pl.kernel,
        mesh=mesh,
        out_type=jax.ShapeDtypeStruct((N, I), jnp.float32),
        scratch_types=[
            pltpu.VMEM((CH,), jnp.int32),
            pltpu.VMEM((CH,), jnp.int32),
            pltpu.VMEM((CH, I), jnp.float32),
            pltpu.VMEM((CH, I), jnp.float32),
            pltpu.SemaphoreType.DMA,
            pltpu.SemaphoreType.DMA,
            pltpu.SemaphoreType.DMA,
            pltpu.SemaphoreType.DMA,
        ],
    )
    def _combine(ys_hbm, pos_hbm, out_hbm, idx0, idx1, rows0, rows1,
                 g0, g1, s0, s1):
        wid = lax.axis_index("s") * 2 + lax.axis_index("c")
        base = wid * TPW
        idx = (idx0, idx1)
        rows = (rows0, rows1)
        gsem = (g0, g1)
        ssem = (s0, s1)
        gathers = [None, None]
        stores = [None, None]
        for c in range(NCH):
            b = c % 2
            if stores[b] is not None:
                stores[b].wait()
            pltpu.sync_copy(pos_hbm.at[pl.ds(base + c * CH, CH)], idx[b])
            gathers[b] = pltpu.async_copy(ys_hbm.at[idx[b]], rows[b], gsem[b])
            o = 1 - b
            if gathers[o] is not None:
                gathers[o].wait()
                off = base + (c - 1) * CH
                stores[o] = pltpu.async_copy(
                    rows[o], out_hbm.at[pl.ds(off, CH)], ssem[o])
                gathers[o] = None
        last = NCH - 1
        b = last % 2
        gathers[b].wait()
        stores[b] = pltpu.async_copy(
            rows[b], out_hbm.at[pl.ds(base + last * CH, CH)], ssem[b])
        for st in stores:
            if st is not None:
                st.wait()

    return _dispatch, _combine


def kernel(x, Wg, We):
    pos2d, te2d, valid2d, row2d = pl.pallas_call(
        _route_kernel,
        out_shape=[
            jax.ShapeDtypeStruct((1, N), jnp.int32),
            jax.ShapeDtypeStruct((1, G), jnp.int32),
            jax.ShapeDtypeStruct((1, G), jnp.int32),
            jax.ShapeDtypeStruct((1, G), jnp.int32),
        ],
    )(x, Wg)
    pos = pos2d.reshape(N)
    te = te2d.reshape(G)
    valid = valid2d.reshape(G)
    row = row2d.reshape(G)

    dispatch, combine = _sc_kernels()
    xs = dispatch(x, pos)

    grid_spec = pltpu.PrefetchScalarGridSpec(
        num_scalar_prefetch=3,
        grid=(G,),
        in_specs=[
            pl.BlockSpec((T, H), lambda g, te, valid, row: (row[g], 0)),
            pl.BlockSpec((1, I, H), lambda g, te, valid, row: (te[g], 0, 0)),
        ],
        out_specs=pl.BlockSpec((T, I), lambda g, te, valid, row: (row[g], 0)),
    )
    ys = pl.pallas_call(
        _mm_kernel,
        grid_spec=grid_spec,
        out_shape=jax.ShapeDtypeStruct((PADDED, I), jnp.float32),
    )(te, valid, row, xs, We)

    return combine(ys, pos)
